# Initial kernel scaffold; baseline (speedup 1.0000x reference)
import jax
import jax.numpy as jnp
from jax.experimental import pallas as pl


def kernel(inputs):
    raise NotImplementedError("write your pallas kernel here")


# speedup vs baseline: 29.9070x; 29.9070x over previous
import functools

import jax
import jax.numpy as jnp
from jax import lax
from jax.experimental import pallas as pl
from jax.experimental.pallas import tpu as pltpu
from jax.experimental.pallas import tpu_sc as plsc

K = 8
L = 16
NC = 2
NS = 16


def _insert(rs, v):
    out = []
    cur = v
    for r in rs:
        out.append(jnp.maximum(r, cur))
        cur = jnp.minimum(r, cur)
    return tuple(out)


def kernel(inputs):
    x = inputs
    B, S, D = x.shape
    NW = NC * NS
    CB = NW // B
    CW = D // CB
    NG = CW // L
    T = 512
    NT = S // T

    mesh = plsc.VectorSubcoreMesh(core_axis_name="c", subcore_axis_name="s")

    @functools.partial(
        pl.kernel,
        out_type=jax.ShapeDtypeStruct((NW, CW * K), jnp.float32),
        mesh=mesh,
        scratch_types=[
            pltpu.VMEM((T, CW), jnp.float32),
            pltpu.VMEM((CW * K,), jnp.float32),
        ],
    )
    def kmax(x_hbm, out_hbm, buf, outv):
        wid = lax.axis_index("s") * NC + lax.axis_index("c")
        b = wid // CB
        c0 = (wid % CB) * CW

        neg = jnp.full((L,), -jnp.inf, jnp.float32)

        def tile_body(t, state):
            pltpu.sync_copy(x_hbm.at[b, pl.ds(t * T, T), pl.ds(c0, CW)], buf)
            new_state = []
            for g in range(NG):
                def s_body(s, rs):
                    return _insert(rs, buf[s, pl.ds(g * L, L)])
                new_state.append(lax.fori_loop(0, T, s_body, state[g]))
            return tuple(new_state)

        init = tuple(tuple(neg for _ in range(K)) for _ in range(NG))
        state = lax.fori_loop(0, NT, tile_body, init)

        for g in range(NG):
            for i in range(K):
                outv[pl.ds(i * CW + g * L, L)] = state[g][i]
        pltpu.sync_copy(outv, out_hbm.at[wid])

    out = kmax(x)
    out = out.reshape(B, CB, K, CW).transpose(0, 1, 3, 2)
    return out.reshape(B, D * K)


# speedup vs baseline: 64.0748x; 2.1425x over previous
import functools

import jax
import jax.numpy as jnp
from jax import lax
from jax.experimental import pallas as pl
from jax.experimental.pallas import tpu as pltpu
from jax.experimental.pallas import tpu_sc as plsc

K = 8
L = 16
NC = 2
NS = 16

_SORT8 = (
    (0, 1), (2, 3), (4, 5), (6, 7),
    (0, 2), (1, 3), (4, 6), (5, 7),
    (1, 2), (5, 6),
    (0, 4), (1, 5), (2, 6), (3, 7),
    (2, 4), (3, 5),
    (1, 2), (3, 4), (5, 6),
)


def _sort8(vs):
    vs = list(vs)
    for i, j in _SORT8:
        hi = jnp.maximum(vs[i], vs[j])
        lo = jnp.minimum(vs[i], vs[j])
        vs[i], vs[j] = hi, lo
    return vs


def _merge_top8(r, v):
    m = [jnp.maximum(r[i], v[7 - i]) for i in range(K)]
    for i, j in ((0, 4), (1, 5), (2, 6), (3, 7),
                 (0, 2), (1, 3), (4, 6), (5, 7),
                 (0, 1), (2, 3), (4, 5), (6, 7)):
        hi = jnp.maximum(m[i], m[j])
        lo = jnp.minimum(m[i], m[j])
        m[i], m[j] = hi, lo
    return tuple(m)


def kernel(inputs):
    x = inputs
    B, S, D = x.shape
    NW = NC * NS
    CB = NW // B
    CW = D // CB
    NG = CW // L
    T = 256
    NT = S // T
    NB = T // K

    mesh = plsc.VectorSubcoreMesh(core_axis_name="c", subcore_axis_name="s")

    @functools.partial(
        pl.kernel,
        out_type=jax.ShapeDtypeStruct((NW, CW * K), jnp.float32),
        mesh=mesh,
        scratch_types=[
            pltpu.VMEM((T, CW), jnp.float32),
            pltpu.VMEM((T, CW), jnp.float32),
            pltpu.VMEM((CW * K,), jnp.float32),
            pltpu.SemaphoreType.DMA,
            pltpu.SemaphoreType.DMA,
        ],
    )
    def kmax(x_hbm, out_hbm, buf0, buf1, outv, sem0, sem1):
        wid = lax.axis_index("s") * NC + lax.axis_index("c")
        b = wid // CB
        c0 = (wid % CB) * CW

        neg = jnp.full((L,), -jnp.inf, jnp.float32)

        def src(t):
            return x_hbm.at[b, pl.ds(t * T, T), pl.ds(c0, CW)]

        def process(buf, state):
            new_state = []
            for g in range(NG):
                def blk_body(blk, rs, _g=g):
                    s0 = blk * K
                    vs = _sort8(buf[s0 + j, pl.ds(_g * L, L)] for j in range(K))
                    return _merge_top8(rs, vs)
                new_state.append(lax.fori_loop(0, NB, blk_body, state[g]))
            return tuple(new_state)

        def pair_body(tt, state):
            t0 = tt * 2
            pltpu.async_copy(src(t0 + 1), buf1, sem1)
            pltpu.make_async_copy(src(t0), buf0, sem0).wait()
            state = process(buf0, state)

            @pl.when(tt + 1 < NT // 2)
            def _():
                pltpu.async_copy(src(t0 + 2), buf0, sem0)

            pltpu.make_async_copy(src(t0 + 1), buf1, sem1).wait()
            return process(buf1, state)

        pltpu.async_copy(src(0), buf0, sem0)
        init = tuple(tuple(neg for _ in range(K)) for _ in range(NG))
        state = lax.fori_loop(0, NT // 2, pair_body, init)

        for g in range(NG):
            for i in range(K):
                outv[pl.ds(i * CW + g * L, L)] = state[g][i]
        pltpu.sync_copy(outv, out_hbm.at[wid])

    out = kmax(x)
    out = out.reshape(B, CB, K, CW).transpose(0, 1, 3, 2)
    return out.reshape(B, D * K)
